# baseline (speedup 1.0000x reference)
import functools

import jax
import jax.numpy as jnp
from jax import lax
from jax.experimental import pallas as pl
from jax.experimental.pallas import tpu as pltpu
from jax.experimental.pallas import tpu_sc as plsc

_NUM_INPUTS = 7
_NUM_VOCAB = 13
_NUM_HIDDEN = 10
_BATCH = 16384
_IDX_STRIDE = 128

_ROWS = 521
_RSTRIDE = 16
_W2_ROW = 520


def _build_table(emb, W1, b1, W2, b2):
    w1r = W1.reshape(_NUM_HIDDEN, _NUM_INPUTS, 3)
    base = jnp.einsum("vd,jpd->pvj", emb, w1r)
    pair01 = (base[0][:, None, :] + base[1][None, :, :]).reshape(169, _NUM_HIDDEN)
    pair23 = (base[2][:, None, :] + base[3][None, :, :]).reshape(169, _NUM_HIDDEN)
    pair45 = (base[4][:, None, :] + base[5][None, :, :]).reshape(169, _NUM_HIDDEN)
    last = base[6] + b1[None, :]
    w2row = jnp.concatenate([W2.reshape(_NUM_HIDDEN), b2.reshape(1),
                             jnp.zeros(5, jnp.float32)])[None, :]
    body = jnp.concatenate([pair01, pair23, pair45, last], axis=0)
    body = jnp.pad(body, ((0, 0), (0, _RSTRIDE - _NUM_HIDDEN)))
    return jnp.concatenate([body, w2row], axis=0).reshape(_ROWS * _RSTRIDE)


def _sc_mlp(idx_pad, tbl_flat, *, rows_per_w):
    blocks = rows_per_w // 16
    mesh = plsc.VectorSubcoreMesh(core_axis_name="c", subcore_axis_name="s")
    info = plsc.get_sparse_core_info()
    num_cores = info.num_cores

    @functools.partial(
        pl.kernel,
        mesh=mesh,
        out_type=jax.ShapeDtypeStruct((_BATCH,), jnp.float32),
        compiler_params=pltpu.CompilerParams(
            needs_layout_passes=False, use_tc_tiling_on_sc=False),
        scratch_types=[
            pltpu.VMEM((rows_per_w, _NUM_INPUTS), jnp.int32),
            pltpu.VMEM((_ROWS * _RSTRIDE,), jnp.float32),
            pltpu.VMEM((rows_per_w,), jnp.float32),
            pltpu.SemaphoreType.DMA,
        ],
    )
    def run(idx_hbm, tbl_hbm, out_hbm, idx_v, tbl_v, out_v, sem):
        wid = lax.axis_index("s") * num_cores + lax.axis_index("c")
        base_row = wid * rows_per_w
        cp1 = pltpu.async_copy(
            idx_hbm.at[pl.ds(base_row, rows_per_w)], idx_v, sem)
        cp2 = pltpu.async_copy(tbl_hbm, tbl_v, sem)
        cp1.wait()
        cp2.wait()

        lanes = lax.iota(jnp.int32, 16)
        w2vec = [
            plsc.load_gather(tbl_v, [jnp.full((16,), _W2_ROW * _RSTRIDE + j, jnp.int32)])
            for j in range(_NUM_HIDDEN)
        ]
        b2vec = plsc.load_gather(
            tbl_v, [jnp.full((16,), _W2_ROW * _RSTRIDE + _NUM_HIDDEN, jnp.int32)])

        @plsc.parallel_loop(0, blocks, unroll=4)
        def body(b):
            rowb = b * 16 + lanes
            iv = [plsc.load_gather(idx_v, [rowb, jnp.full((16,), p, jnp.int32)])
                  for p in range(_NUM_INPUTS)]
            k0 = iv[0] * (13 * _RSTRIDE) + iv[1] * _RSTRIDE
            k1 = iv[2] * (13 * _RSTRIDE) + iv[3] * _RSTRIDE + 169 * _RSTRIDE
            k2 = iv[4] * (13 * _RSTRIDE) + iv[5] * _RSTRIDE + 338 * _RSTRIDE
            k3 = iv[6] * _RSTRIDE + 507 * _RSTRIDE
            acc = b2vec
            for j in range(_NUM_HIDDEN):
                h = (plsc.load_gather(tbl_v, [k0 + j])
                     + plsc.load_gather(tbl_v, [k1 + j])
                     + plsc.load_gather(tbl_v, [k2 + j])
                     + plsc.load_gather(tbl_v, [k3 + j]))
                acc = acc + w2vec[j] * jnp.maximum(h, 0.0)
            out_v[pl.ds(b * 16, 16)] = acc

        pltpu.sync_copy(out_v, out_hbm.at[pl.ds(base_row, rows_per_w)])

    return run(idx_pad, tbl_flat)


def kernel(input, emb, W1, b1, W2, b2):
    info = plsc.get_sparse_core_info()
    n_workers = info.num_cores * info.num_subcores
    rows_per_w = _BATCH // n_workers
    tbl = _build_table(emb, W1, b1, W2, b2)
    return _sc_mlp(input.astype(jnp.int32), tbl, rows_per_w=rows_per_w)


# speedup vs baseline: 1.2732x; 1.2732x over previous
import functools

import jax
import jax.numpy as jnp
from jax import lax
from jax.experimental import pallas as pl
from jax.experimental.pallas import tpu as pltpu
from jax.experimental.pallas import tpu_sc as plsc

_NUM_INPUTS = 7
_NUM_VOCAB = 13
_NUM_HIDDEN = 10
_BATCH = 16384
_IDX_STRIDE = 128

_ROWS = 521
_RSTRIDE = 17
_W2_ROW = 520


def _build_table(emb, W1, b1, W2, b2):
    w1r = W1.reshape(_NUM_HIDDEN, _NUM_INPUTS, 3)
    base = jnp.einsum("vd,jpd->pvj", emb, w1r)
    pair01 = (base[0][:, None, :] + base[1][None, :, :]).reshape(169, _NUM_HIDDEN)
    pair23 = (base[2][:, None, :] + base[3][None, :, :]).reshape(169, _NUM_HIDDEN)
    pair45 = (base[4][:, None, :] + base[5][None, :, :]).reshape(169, _NUM_HIDDEN)
    last = base[6] + b1[None, :]
    w2row = jnp.concatenate([W2.reshape(_NUM_HIDDEN), b2.reshape(1),
                             jnp.zeros(_RSTRIDE - _NUM_HIDDEN - 1,
                                       jnp.float32)])[None, :]
    body = jnp.concatenate([pair01, pair23, pair45, last], axis=0)
    body = jnp.pad(body, ((0, 0), (0, _RSTRIDE - _NUM_HIDDEN)))
    return jnp.concatenate([body, w2row], axis=0).reshape(_ROWS * _RSTRIDE)


def _sc_mlp(idx_pad, tbl_flat, *, rows_per_w):
    blocks = rows_per_w // 16
    mesh = plsc.VectorSubcoreMesh(core_axis_name="c", subcore_axis_name="s")
    info = plsc.get_sparse_core_info()
    num_cores = info.num_cores

    @functools.partial(
        pl.kernel,
        mesh=mesh,
        out_type=jax.ShapeDtypeStruct((_BATCH,), jnp.float32),
        compiler_params=pltpu.CompilerParams(needs_layout_passes=False),
        scratch_types=[
            pltpu.VMEM((rows_per_w, _NUM_INPUTS), jnp.int32),
            pltpu.VMEM((_ROWS * _RSTRIDE,), jnp.float32),
            pltpu.VMEM((rows_per_w,), jnp.float32),
            pltpu.SemaphoreType.DMA,
        ],
    )
    def run(idx_hbm, tbl_hbm, out_hbm, idx_v, tbl_v, out_v, sem):
        wid = lax.axis_index("s") * num_cores + lax.axis_index("c")
        base_row = wid * rows_per_w
        cp1 = pltpu.async_copy(
            idx_hbm.at[pl.ds(base_row, rows_per_w)], idx_v, sem)
        cp2 = pltpu.async_copy(tbl_hbm, tbl_v, sem)
        cp1.wait()
        cp2.wait()

        lanes = lax.iota(jnp.int32, 16)
        w2vec = [
            plsc.load_gather(tbl_v, [jnp.full((16,), _W2_ROW * _RSTRIDE + j, jnp.int32)])
            for j in range(_NUM_HIDDEN)
        ]
        b2vec = plsc.load_gather(
            tbl_v, [jnp.full((16,), _W2_ROW * _RSTRIDE + _NUM_HIDDEN, jnp.int32)])

        @plsc.parallel_loop(0, blocks, unroll=4)
        def body(b):
            rowb = b * 16 + lanes
            iv = [plsc.load_gather(idx_v, [rowb, jnp.full((16,), p, jnp.int32)])
                  for p in range(_NUM_INPUTS)]
            k0 = iv[0] * (13 * _RSTRIDE) + iv[1] * _RSTRIDE
            k1 = iv[2] * (13 * _RSTRIDE) + iv[3] * _RSTRIDE + 169 * _RSTRIDE
            k2 = iv[4] * (13 * _RSTRIDE) + iv[5] * _RSTRIDE + 338 * _RSTRIDE
            k3 = iv[6] * _RSTRIDE + 507 * _RSTRIDE
            acc = b2vec
            for j in range(_NUM_HIDDEN):
                h = (plsc.load_gather(tbl_v, [k0 + j])
                     + plsc.load_gather(tbl_v, [k1 + j])
                     + plsc.load_gather(tbl_v, [k2 + j])
                     + plsc.load_gather(tbl_v, [k3 + j]))
                acc = acc + w2vec[j] * jnp.maximum(h, 0.0)
            out_v[pl.ds(b * 16, 16)] = acc

        pltpu.sync_copy(out_v, out_hbm.at[pl.ds(base_row, rows_per_w)])

    return run(idx_pad, tbl_flat)


def kernel(input, emb, W1, b1, W2, b2):
    info = plsc.get_sparse_core_info()
    n_workers = info.num_cores * info.num_subcores
    rows_per_w = _BATCH // n_workers
    tbl = _build_table(emb, W1, b1, W2, b2)
    return _sc_mlp(input.astype(jnp.int32), tbl, rows_per_w=rows_per_w)


# speedup vs baseline: 1.7183x; 1.3496x over previous
import functools

import jax
import jax.numpy as jnp
from jax import lax
from jax.experimental import pallas as pl
from jax.experimental.pallas import tpu as pltpu
from jax.experimental.pallas import tpu_sc as plsc

_NUM_INPUTS = 7
_NUM_VOCAB = 13
_NUM_HIDDEN = 10
_BATCH = 16384

_ROWS = 521
_RSTRIDE = 17
_W2_ROW = 520


def _build_table(emb, W1, b1, W2, b2):
    w1r = W1.reshape(_NUM_HIDDEN, _NUM_INPUTS, 3)
    base = jnp.einsum("vd,jpd->pvj", emb, w1r)
    pair01 = (base[0][:, None, :] + base[1][None, :, :]).reshape(169, _NUM_HIDDEN)
    pair23 = (base[2][:, None, :] + base[3][None, :, :]).reshape(169, _NUM_HIDDEN)
    pair45 = (base[4][:, None, :] + base[5][None, :, :]).reshape(169, _NUM_HIDDEN)
    last = base[6] + b1[None, :]
    w2row = jnp.concatenate([W2.reshape(_NUM_HIDDEN), b2.reshape(1),
                             jnp.zeros(_RSTRIDE - _NUM_HIDDEN - 1,
                                       jnp.float32)])[None, :]
    body = jnp.concatenate([pair01, pair23, pair45, last], axis=0)
    body = jnp.pad(body, ((0, 0), (0, _RSTRIDE - _NUM_HIDDEN)))
    return jnp.concatenate([body, w2row], axis=0).reshape(_ROWS * _RSTRIDE)


def _pack_keys(input):
    iv = input.astype(jnp.int32)
    q0 = iv[:, 0] * _NUM_VOCAB + iv[:, 1]
    q1 = iv[:, 2] * _NUM_VOCAB + iv[:, 3]
    q2 = iv[:, 4] * _NUM_VOCAB + iv[:, 5]
    q3 = iv[:, 6]
    return q0 | (q1 << 8) | (q2 << 16) | (q3 << 24)


def _sc_mlp(keys_packed, tbl_flat, *, rows_per_w):
    blocks = rows_per_w // 16
    mesh = plsc.VectorSubcoreMesh(core_axis_name="c", subcore_axis_name="s")
    info = plsc.get_sparse_core_info()
    num_cores = info.num_cores

    @functools.partial(
        pl.kernel,
        mesh=mesh,
        out_type=jax.ShapeDtypeStruct((_BATCH,), jnp.float32),
        compiler_params=pltpu.CompilerParams(needs_layout_passes=False),
        scratch_types=[
            pltpu.VMEM((rows_per_w,), jnp.int32),
            pltpu.VMEM((_ROWS * _RSTRIDE,), jnp.float32),
            pltpu.VMEM((rows_per_w,), jnp.float32),
            pltpu.SemaphoreType.DMA,
        ],
    )
    def run(keys_hbm, tbl_hbm, out_hbm, keys_v, tbl_v, out_v, sem):
        wid = lax.axis_index("s") * num_cores + lax.axis_index("c")
        base_row = wid * rows_per_w
        cp1 = pltpu.async_copy(keys_hbm.at[pl.ds(base_row, rows_per_w)], keys_v, sem)
        cp2 = pltpu.async_copy(tbl_hbm, tbl_v, sem)
        cp1.wait()
        cp2.wait()

        w2vec = [
            plsc.load_gather(tbl_v, [jnp.full((16,), _W2_ROW * _RSTRIDE + j, jnp.int32)])
            for j in range(_NUM_HIDDEN)
        ]
        b2vec = plsc.load_gather(
            tbl_v, [jnp.full((16,), _W2_ROW * _RSTRIDE + _NUM_HIDDEN, jnp.int32)])

        @plsc.parallel_loop(0, blocks, unroll=4)
        def body(b):
            w = keys_v[pl.ds(b * 16, 16)]
            k0 = (w & 255) * _RSTRIDE
            k1 = ((w >> 8) & 255) * _RSTRIDE + 169 * _RSTRIDE
            k2 = ((w >> 16) & 255) * _RSTRIDE + 338 * _RSTRIDE
            k3 = (w >> 24) * _RSTRIDE + 507 * _RSTRIDE
            acc = b2vec
            for j in range(_NUM_HIDDEN):
                h = (plsc.load_gather(tbl_v, [k0 + j])
                     + plsc.load_gather(tbl_v, [k1 + j])
                     + plsc.load_gather(tbl_v, [k2 + j])
                     + plsc.load_gather(tbl_v, [k3 + j]))
                acc = acc + w2vec[j] * jnp.maximum(h, 0.0)
            out_v[pl.ds(b * 16, 16)] = acc

        pltpu.sync_copy(out_v, out_hbm.at[pl.ds(base_row, rows_per_w)])

    return run(keys_packed, tbl_flat)


def kernel(input, emb, W1, b1, W2, b2):
    info = plsc.get_sparse_core_info()
    n_workers = info.num_cores * info.num_subcores
    rows_per_w = _BATCH // n_workers
    tbl = _build_table(emb, W1, b1, W2, b2)
    keys = _pack_keys(input)
    return _sc_mlp(keys, tbl, rows_per_w=rows_per_w)
